# baseline (speedup 1.0000x reference)
import functools

import jax
import jax.numpy as jnp
from jax import lax
from jax.experimental import pallas as pl
from jax.experimental.pallas import tpu as pltpu
from jax.experimental.pallas import tpu_sc as plsc

_B, _N, _G, _M = 8, 8192, 512, 32
_GT = 128
_PAD_D = 8
_ROWS = _B * _G * _M
_NW = 32
_RPW = _ROWS // _NW
_CHUNK = 1024


_NCH = 64
_K0 = 6


def _topk_body(b0, c_ref, p_ref, idx_ref, sval_ref, sidx_ref):
    b = pl.program_id(0) + b0
    c = c_ref[0]
    pT = p_ref[0]
    cn = jnp.transpose(
        jnp.sum(c * c, axis=1, keepdims=True))
    pn = jnp.transpose(
        jnp.sum(pT * pT, axis=0, keepdims=True))
    cT = jnp.transpose(c)
    cpT = lax.dot_general(pT, cT, (((0,), (0,)), ((), ())),
                          preferred_element_type=jnp.float32)
    dT = cn + pn - 2.0 * cpT

    wiota = lax.broadcasted_iota(
        jnp.int32, (128, _GT), 0).astype(jnp.float32)
    miota = lax.broadcasted_iota(jnp.int32, (_M, _GT), 0)
    inf = jnp.float32(jnp.inf)

    for ci in range(_NCH):
        dd = dT[ci * 128:(ci + 1) * 128, :]
        for t in range(_K0):
            cm = jnp.min(dd, axis=0, keepdims=True)
            lidx = jnp.min(jnp.where(dd == cm, wiota, 128.0),
                           axis=0, keepdims=True)
            sval_ref[t, ci] = cm[0]
            sidx_ref[t, ci] = lidx[0] + (ci * 128.0)
            if t + 1 < _K0:
                dd = jnp.where(wiota == lidx, inf, dd)

    ciota = lax.broadcasted_iota(
        jnp.int32, (_NCH, _GT), 0).astype(jnp.float32)
    svalL = sval_ref[_K0 - 1]
    sidxL = sidx_ref[_K0 - 1]

    svals = [sval_ref[t] for t in range(_K0 - 1)]
    sidxs = [sidx_ref[t] for t in range(_K0 - 1)]
    pc = jnp.zeros((_NCH, _GT), jnp.int32)
    pops = []
    for i in range(_M):
        head, ihead = svalL, sidxL
        for t in range(_K0 - 1):
            sel = pc == t
            head = jnp.where(sel, svals[t], head)
            ihead = jnp.where(sel, sidxs[t], ihead)
        m = jnp.min(head, axis=0)
        cstar = jnp.min(jnp.where(head == m[None, :], ciota, float(_NCH)),
                        axis=0)
        selmask = ciota == cstar[None, :]
        pops.append(jnp.max(jnp.where(selmask, ihead, -1.0), axis=0))
        pc = pc + selmask.astype(jnp.int32)
    acc = jnp.stack(pops, axis=0)
    acc0 = jnp.zeros((_M, _GT), jnp.float32)
    bad = jnp.max(pc) > _K0

    def slow():
        piota = lax.broadcasted_iota(
            jnp.int32, (_N, _GT), 0).astype(jnp.float32)

        def body(j, st):
            d, a = st
            mm = jnp.min(d, axis=0, keepdims=True)
            am = jnp.min(jnp.where(d == mm, piota, float(_N)),
                         axis=0, keepdims=True)
            d = jnp.where(piota == am, inf, d)
            a = jnp.where(miota == j, am, a)
            return d, a

        _, a = lax.fori_loop(0, _M, body, (dT, acc0))
        return a

    res = lax.cond(bad, slow, lambda: acc)
    idx_ref[0, 0] = res.astype(jnp.int32) + b * _N


def _topk_indices(center_pad, ptsT, b0, nb):
    return pl.pallas_call(
        functools.partial(_topk_body, b0),
        grid=(nb, _G // _GT),
        in_specs=[
            pl.BlockSpec((1, _GT, _PAD_D), lambda b, g: (b0 + b, g, 0)),
            pl.BlockSpec((1, _PAD_D, _N), lambda b, g: (b0 + b, 0, 0)),
        ],
        out_specs=pl.BlockSpec((1, 1, _M, _GT), lambda b, g: (b, g, 0, 0)),
        out_shape=jax.ShapeDtypeStruct((nb, _G // _GT, _M, _GT), jnp.int32),
        scratch_shapes=[
            pltpu.VMEM((_K0, _NCH, _GT), jnp.float32),
            pltpu.VMEM((_K0, _NCH, _GT), jnp.float32),
        ],
    )(center_pad, ptsT)


_HROWS = _ROWS // 2
_RPW = _HROWS // _NW
_NGRP = _RPW // _M


def _gather_sub(tab, ctab, idx, half):
    mesh = plsc.VectorSubcoreMesh(core_axis_name="c", subcore_axis_name="s")

    @functools.partial(
        pl.kernel, mesh=mesh,
        compiler_params=pltpu.CompilerParams(use_tc_tiling_on_sc=False),
        out_type=jax.ShapeDtypeStruct((_HROWS, 16), jnp.float32),
        scratch_types=[
            pltpu.VMEM((_RPW,), jnp.int32),
            pltpu.VMEM((_NGRP,), jnp.int32),
            pltpu.VMEM((_RPW, 16), jnp.float32),
            pltpu.VMEM((_NGRP, 16), jnp.float32),
            pltpu.SemaphoreType.DMA,
            pltpu.SemaphoreType.DMA,
        ],
    )
    def k(tab_hbm, ctab_hbm, idx_hbm, out_hbm,
          idxv, cidxv, ptsv, ctrv, sem1, sem2):
        wid = lax.axis_index("s") * 2 + lax.axis_index("c")
        base = wid * _RPW
        cbase = half * (_HROWS // _M) + wid * _NGRP
        for kk in range(_NGRP // 16):
            cidxv[pl.ds(kk * 16, 16)] = (
                cbase + kk * 16 + lax.iota(jnp.int32, 16))
        pltpu.sync_copy(idx_hbm.at[pl.ds(base, _RPW)], idxv)
        cp1 = pltpu.async_copy(tab_hbm.at[idxv], ptsv, sem1)
        cp2 = pltpu.async_copy(ctab_hbm.at[cidxv], ctrv, sem2)
        cp1.wait()
        cp2.wait()

        def sub(grp, carry):
            ctr = ctrv[grp, :]
            for j in range(_M):
                r = grp * _M + j
                ptsv[r, :] = ptsv[r, :] - ctr
            return carry

        lax.fori_loop(0, _NGRP, sub, 0)
        pltpu.sync_copy(ptsv, out_hbm.at[pl.ds(base, _RPW)])

    return k(tab, ctab, idx)


def kernel(xyz, center):
    pts = xyz[..., :3]
    zpad = jnp.zeros((_B, _N, _PAD_D - 3), jnp.float32)
    pts_pad = jnp.concatenate([pts, zpad], axis=-1)
    ptsT = pts_pad.transpose(0, 2, 1)
    cpad = jnp.concatenate(
        [center, jnp.zeros((_B, _G, _PAD_D - 3), jnp.float32)],
        axis=-1)
    tab = jnp.pad(xyz.reshape(_B * _N, 6), ((0, 0), (0, 10)))
    ctab = jnp.pad(center.reshape(_B * _G, 3), ((0, 0), (0, 13)))
    hb = _B // 2
    outs = []
    for h in range(2):
        idx4 = _topk_indices(cpad, ptsT, h * hb, hb)
        idx = idx4.transpose(0, 1, 3, 2).reshape(-1)
        outs.append(_gather_sub(tab, ctab, idx, h))
    out = jnp.concatenate(outs, axis=0)
    return out.reshape(_B, _G, _M, 16)[..., :6]


# speedup vs baseline: 1.1977x; 1.1977x over previous
import functools

import jax
import jax.numpy as jnp
from jax import lax
from jax.experimental import pallas as pl
from jax.experimental.pallas import tpu as pltpu
from jax.experimental.pallas import tpu_sc as plsc

_B, _N, _G, _M = 8, 8192, 512, 32
_GT = 256
_PAD_D = 8
_ROWS = _B * _G * _M
_NW = 32
_RPW = _ROWS // _NW
_CHUNK = 1024


_NCH = 64
_K0 = 6


def _topk_body(c_ref, p_ref, idx_ref, sval_ref, sidx_ref):
    b = pl.program_id(0)
    c = c_ref[0]
    pT = p_ref[0]
    cn = jnp.transpose(
        jnp.sum(c * c, axis=1, keepdims=True))
    pn = jnp.transpose(
        jnp.sum(pT * pT, axis=0, keepdims=True))
    cT = jnp.transpose(c)
    cpT = lax.dot_general(pT, cT, (((0,), (0,)), ((), ())),
                          preferred_element_type=jnp.float32)
    dT = cn + pn - 2.0 * cpT

    wiota = lax.broadcasted_iota(
        jnp.int32, (128, _GT), 0).astype(jnp.float32)
    miota = lax.broadcasted_iota(jnp.int32, (_M, _GT), 0)
    inf = jnp.float32(jnp.inf)

    for ci in range(_NCH):
        dd = dT[ci * 128:(ci + 1) * 128, :]
        for t in range(_K0):
            cm = jnp.min(dd, axis=0, keepdims=True)
            lidx = jnp.min(jnp.where(dd == cm, wiota, 128.0),
                           axis=0, keepdims=True)
            sval_ref[t, ci] = cm[0]
            sidx_ref[t, ci] = lidx[0] + (ci * 128.0)
            if t + 1 < _K0:
                dd = jnp.where(wiota == lidx, inf, dd)

    ciota = lax.broadcasted_iota(
        jnp.int32, (_NCH, _GT), 0).astype(jnp.float32)
    svalL = sval_ref[_K0 - 1]
    sidxL = sidx_ref[_K0 - 1]

    svals = [sval_ref[t] for t in range(_K0 - 1)]
    sidxs = [sidx_ref[t] for t in range(_K0 - 1)]
    pc = jnp.zeros((_NCH, _GT), jnp.int32)
    pops = []
    for i in range(_M):
        head, ihead = svalL, sidxL
        for t in range(_K0 - 1):
            sel = pc == t
            head = jnp.where(sel, svals[t], head)
            ihead = jnp.where(sel, sidxs[t], ihead)
        m = jnp.min(head, axis=0)
        cstar = jnp.min(jnp.where(head == m[None, :], ciota, float(_NCH)),
                        axis=0)
        selmask = ciota == cstar[None, :]
        pops.append(jnp.max(jnp.where(selmask, ihead, -1.0), axis=0))
        pc = pc + selmask.astype(jnp.int32)
    acc = jnp.stack(pops, axis=0)
    acc0 = jnp.zeros((_M, _GT), jnp.float32)
    bad = jnp.max(pc) > _K0

    def slow():
        piota = lax.broadcasted_iota(
            jnp.int32, (_N, _GT), 0).astype(jnp.float32)

        def body(j, st):
            d, a = st
            mm = jnp.min(d, axis=0, keepdims=True)
            am = jnp.min(jnp.where(d == mm, piota, float(_N)),
                         axis=0, keepdims=True)
            d = jnp.where(piota == am, inf, d)
            a = jnp.where(miota == j, am, a)
            return d, a

        _, a = lax.fori_loop(0, _M, body, (dT, acc0))
        return a

    res = lax.cond(bad, slow, lambda: acc)
    idx_ref[0, 0] = res.astype(jnp.int32) + b * _N


def _topk_indices(center_pad, ptsT):
    return pl.pallas_call(
        _topk_body,
        grid=(_B, _G // _GT),
        in_specs=[
            pl.BlockSpec((1, _GT, _PAD_D), lambda b, g: (b, g, 0)),
            pl.BlockSpec((1, _PAD_D, _N), lambda b, g: (b, 0, 0)),
        ],
        out_specs=pl.BlockSpec((1, 1, _M, _GT), lambda b, g: (b, g, 0, 0)),
        out_shape=jax.ShapeDtypeStruct((_B, _G // _GT, _M, _GT), jnp.int32),
        scratch_shapes=[
            pltpu.VMEM((_K0, _NCH, _GT), jnp.float32),
            pltpu.VMEM((_K0, _NCH, _GT), jnp.float32),
        ],
    )(center_pad, ptsT)


_NGRP = _RPW // _M


def _gather_sub(tab, ctab, idx):
    mesh = plsc.VectorSubcoreMesh(core_axis_name="c", subcore_axis_name="s")

    @functools.partial(
        pl.kernel, mesh=mesh,
        compiler_params=pltpu.CompilerParams(use_tc_tiling_on_sc=False),
        out_type=jax.ShapeDtypeStruct((_ROWS, 16), jnp.float32),
        scratch_types=[
            pltpu.VMEM((_RPW,), jnp.int32),
            pltpu.VMEM((_NGRP,), jnp.int32),
            pltpu.VMEM((_RPW, 16), jnp.float32),
            pltpu.VMEM((_NGRP, 16), jnp.float32),
            pltpu.SemaphoreType.DMA,
            pltpu.SemaphoreType.DMA,
        ],
    )
    def k(tab_hbm, ctab_hbm, idx_hbm, out_hbm,
          idxv, cidxv, ptsv, ctrv, sem1, sem2):
        wid = lax.axis_index("s") * 2 + lax.axis_index("c")
        base = wid * _RPW
        cbase = wid * _NGRP
        for kk in range(_NGRP // 16):
            cidxv[pl.ds(kk * 16, 16)] = (
                cbase + kk * 16 + lax.iota(jnp.int32, 16))
        pltpu.sync_copy(idx_hbm.at[pl.ds(base, _RPW)], idxv)
        cp1 = pltpu.async_copy(tab_hbm.at[idxv], ptsv, sem1)
        cp2 = pltpu.async_copy(ctab_hbm.at[cidxv], ctrv, sem2)
        cp1.wait()
        cp2.wait()

        def sub(grp, carry):
            ctr = ctrv[grp, :]
            for j in range(_M):
                r = grp * _M + j
                ptsv[r, :] = ptsv[r, :] - ctr
            return carry

        lax.fori_loop(0, _NGRP, sub, 0)
        pltpu.sync_copy(ptsv, out_hbm.at[pl.ds(base, _RPW)])

    return k(tab, ctab, idx)


def kernel(xyz, center):
    pts = xyz[..., :3]
    zpad = jnp.zeros((_B, _N, _PAD_D - 3), jnp.float32)
    pts_pad = jnp.concatenate([pts, zpad], axis=-1)
    ptsT = pts_pad.transpose(0, 2, 1)
    cpad = jnp.concatenate(
        [center, jnp.zeros((_B, _G, _PAD_D - 3), jnp.float32)],
        axis=-1)
    idx4 = _topk_indices(cpad, ptsT)
    idx = idx4.transpose(0, 1, 3, 2).reshape(_B, _G, _M)
    tab = jnp.pad(xyz.reshape(_B * _N, 6), ((0, 0), (0, 10)))
    ctab = jnp.pad(center.reshape(_B * _G, 3), ((0, 0), (0, 13)))
    out = _gather_sub(tab, ctab, idx.reshape(-1))
    return out.reshape(_B, _G, _M, 16)[..., :6]
